# Initial kernel scaffold; baseline (speedup 1.0000x reference)
import jax
import jax.numpy as jnp
from jax.experimental import pallas as pl


def kernel(x, edge_index, W1, b1, W2, b2, w3, b3):
    raise NotImplementedError("write your pallas kernel here")


# speedup vs baseline: 708.0415x; 708.0415x over previous
import functools

import jax
import jax.numpy as jnp
from jax import lax
from jax.experimental import pallas as pl
from jax.experimental.pallas import tpu as pltpu
from jax.experimental.pallas import tpu_sc as plsc

CAP = 80
ROWS_PER_SC = 320
ROW_BLK = 80
NW = 32

_SC_PARAMS = pltpu.CompilerParams(needs_layout_passes=False)


def _leaky(t):
    return jnp.maximum(t, 0.01 * t)


def _encode_kernel(x_ref, w1_ref, b1_ref, w2_ref, b2_ref, xe_ref, y_ref, yd_ref):
    h = jnp.dot(x_ref[...], w1_ref[...], preferred_element_type=jnp.float32)
    h = h + b1_ref[...]
    xe = _leaky(h)
    xe_ref[...] = xe
    y = jnp.dot(xe, w2_ref[...], preferred_element_type=jnp.float32)
    y_ref[...] = y
    yd_ref[...] = y - b2_ref[...]


def _encode(x, W1, b1, W2, b2):
    N, d_in = x.shape
    d_lat = W1.shape[1]
    blk = 400
    grid = (N // blk,)
    return pl.pallas_call(
        _encode_kernel,
        grid=grid,
        in_specs=[
            pl.BlockSpec((blk, d_in), lambda i: (i, 0)),
            pl.BlockSpec((d_in, d_lat), lambda i: (0, 0)),
            pl.BlockSpec((1, d_lat), lambda i: (0, 0)),
            pl.BlockSpec((d_lat, d_lat), lambda i: (0, 0)),
            pl.BlockSpec((1, d_lat), lambda i: (0, 0)),
        ],
        out_specs=[
            pl.BlockSpec((blk, d_lat), lambda i: (i, 0)),
            pl.BlockSpec((blk, d_lat), lambda i: (i, 0)),
            pl.BlockSpec((blk, d_lat), lambda i: (i, 0)),
        ],
        out_shape=[
            jax.ShapeDtypeStruct((N, d_lat), jnp.float32),
            jax.ShapeDtypeStruct((N, d_lat), jnp.float32),
            jax.ShapeDtypeStruct((N, d_lat), jnp.float32),
        ],
    )(x, W1, b1.reshape(1, -1), W2, b2.reshape(1, -1))


BCAP = 512
SENT = 0x7FFF0000
CHUNK = 80


def _vreg_group_slots(key, iota, tmp_v):
    ks, perm = plsc.sort_key_val(key, iota)
    tmp_v[...] = ks
    prv = plsc.load_gather(tmp_v, [jnp.maximum(iota - 1, 0)])
    nxt = plsc.load_gather(tmp_v, [jnp.minimum(iota + 1, 15)])
    new = (iota == 0) | (ks != prv)
    m0 = plsc.cummax(jnp.where(new, iota, 0))
    p = iota - m0
    last = (iota == 15) | (ks != nxt)
    return ks, perm, p, last


def _edge_bucket(y, yd, src, dst):
    N, d_lat = y.shape
    E = src.shape[0]
    epw = E // NW
    nchunks = epw // CHUNK

    mesh = plsc.VectorSubcoreMesh(core_axis_name="c", subcore_axis_name="s")

    @functools.partial(
        pl.kernel,
        out_type=(jax.ShapeDtypeStruct((NW, NW, BCAP), jnp.int32),
                  jax.ShapeDtypeStruct((NW, NW, BCAP), jnp.int32),
                  jax.ShapeDtypeStruct((NW, NW, BCAP), jnp.float32)),
        mesh=mesh,
        compiler_params=_SC_PARAMS,
        scratch_types=[
            pltpu.VMEM((CHUNK,), jnp.int32),
            pltpu.VMEM((CHUNK,), jnp.int32),
            pltpu.VMEM((CHUNK, 128), jnp.float32),
            pltpu.VMEM((CHUNK, 128), jnp.float32),
            pltpu.VMEM((CHUNK,), jnp.int32),
            pltpu.VMEM((CHUNK,), jnp.int32),
            pltpu.VMEM((CHUNK, 128), jnp.float32),
            pltpu.VMEM((CHUNK, 128), jnp.float32),
            pltpu.VMEM((CHUNK,), jnp.float32),
            pltpu.VMEM((NW, BCAP), jnp.int32),
            pltpu.VMEM((NW, BCAP), jnp.int32),
            pltpu.VMEM((NW, BCAP), jnp.float32),
            pltpu.VMEM((NW,), jnp.int32),
            pltpu.VMEM((16,), jnp.int32),
            pltpu.SemaphoreType.DMA,
            pltpu.SemaphoreType.DMA,
            pltpu.SemaphoreType.DMA,
            pltpu.SemaphoreType.DMA,
        ],
    )
    def kern(src_hbm, dst_hbm, y_hbm, yd_hbm, bs_hbm, bd_hbm, bv_hbm,
             src_v0, dst_v0, ysrc_v0, ydst_v0, src_v1, dst_v1, ysrc_v1, ydst_v1,
             sbuf, bS_v, bD_v, bV_v, cnt_v,
             tmp_v, sem_a, sem_b, sem_g0, sem_g1):
        wid = lax.axis_index("s") * 2 + lax.axis_index("c")
        ebase = wid * epw
        iota = lax.iota(jnp.int32, 16)
        sent16 = jnp.full((16,), SENT, jnp.int32)
        izero16 = jnp.zeros((16,), jnp.int32)

        def _prefill(i, _):
            r = i // (BCAP // 16)
            c = i % (BCAP // 16)
            bS_v[r, pl.ds(c * 16, 16)] = sent16
            return 0
        lax.fori_loop(0, NW * (BCAP // 16), _prefill, 0)
        cnt_v[pl.ds(0, 16)] = izero16
        cnt_v[pl.ds(16, 16)] = izero16

        bufs = ((src_v0, dst_v0, ysrc_v0, ydst_v0, sem_g0),
                (src_v1, dst_v1, ysrc_v1, ydst_v1, sem_g1))

        def issue(c, p):
            src_v, dst_v, ysrc_v, ydst_v, sem_g = bufs[p]
            off = ebase + c * CHUNK
            h1 = pltpu.async_copy(src_hbm.at[pl.ds(off, CHUNK)], src_v, sem_a)
            h2 = pltpu.async_copy(dst_hbm.at[pl.ds(off, CHUNK)], dst_v, sem_b)
            h1.wait()
            h2.wait()
            pltpu.async_copy(y_hbm.at[src_v], ysrc_v, sem_g)
            pltpu.async_copy(yd_hbm.at[dst_v], ydst_v, sem_g)

        def compute(p):
            src_v, dst_v, ysrc_v, ydst_v, sem_g = bufs[p]
            pltpu.make_async_copy(y_hbm.at[src_v], ysrc_v, sem_g).wait()
            pltpu.make_async_copy(yd_hbm.at[dst_v], ydst_v, sem_g).wait()

            def edge_pair(e2, _):
                for half in range(2):
                    e = e2 * 2 + half
                    acc = jnp.zeros((16,), jnp.float32)
                    for cc in range(d_lat // 16):
                        a = ysrc_v[e, pl.ds(cc * 16, 16)]
                        b = ydst_v[e, pl.ds(cc * 16, 16)]
                        t = a - b
                        acc = acc + jnp.maximum(t, 0.01 * t)
                    s = jnp.sum(acc)
                    plsc.store_scatter(
                        sbuf, [jnp.full((16,), e, jnp.int32)],
                        jnp.full((16,), s), mask=(iota == 0))
                return 0
            lax.fori_loop(0, CHUNK // 2, edge_pair, 0)

            for g in range(CHUNK // 16):
                raw = sbuf[pl.ds(g * 16, 16)]
                v16 = 1.0 / (1.0 + jnp.exp(-raw))
                sbuf[pl.ds(g * 16, 16)] = v16
                s16 = src_v[pl.ds(g * 16, 16)]
                b16 = s16 // ROWS_PER_SC
                ks, perm, p_, last = _vreg_group_slots(b16, iota, tmp_v)
                cold = plsc.load_gather(cnt_v, [ks])
                slot = cold + p_
                ok = slot < BCAP
                plsc.store_scatter(cnt_v, [ks], jnp.minimum(slot + 1, BCAP),
                                   mask=last)
                pabs = perm + g * 16
                sp = plsc.load_gather(src_v, [pabs])
                dp = plsc.load_gather(dst_v, [pabs])
                vp = plsc.load_gather(sbuf, [pabs])
                plsc.store_scatter(bS_v, [ks, slot], sp, mask=ok)
                plsc.store_scatter(bD_v, [ks, slot], dp, mask=ok)
                plsc.store_scatter(bV_v, [ks, slot], vp, mask=ok)

        issue(0, 0)

        def pair_body(i, _):
            issue(2 * i + 1, 1)
            compute(0)
            issue(2 * i + 2, 0)
            compute(1)
            return 0
        lax.fori_loop(0, (nchunks - 1) // 2, pair_body, 0)
        compute(0)

        pltpu.sync_copy(bS_v, bs_hbm.at[wid])
        pltpu.sync_copy(bD_v, bd_hbm.at[wid])
        pltpu.sync_copy(bV_v, bv_hbm.at[wid])

    return kern(src, dst, y, yd)


def _build_tables(bS, bD, bV, n_pad):
    mesh = plsc.VectorSubcoreMesh(core_axis_name="c", subcore_axis_name="s")

    @functools.partial(
        pl.kernel,
        out_type=(jax.ShapeDtypeStruct((n_pad, CAP), jnp.float32),
                  jax.ShapeDtypeStruct((n_pad, CAP), jnp.int32)),
        mesh=mesh,
        compiler_params=_SC_PARAMS,
        scratch_types=[
            pltpu.VMEM((ROWS_PER_SC, CAP), jnp.float32),
            pltpu.VMEM((ROWS_PER_SC, CAP), jnp.int32),
            pltpu.VMEM((ROWS_PER_SC,), jnp.int32),
            pltpu.VMEM((BCAP,), jnp.int32),
            pltpu.VMEM((BCAP,), jnp.int32),
            pltpu.VMEM((BCAP,), jnp.float32),
            pltpu.VMEM((16,), jnp.int32),
            pltpu.SemaphoreType.DMA,
        ],
    )
    def kern(bs_hbm, bd_hbm, bv_hbm, vt_hbm, dt_hbm,
             Vt_v, Dt_v, rcnt_v, eS_v, eD_v, eV_v, tmp_v, sem):
        wid = lax.axis_index("s") * 2 + lax.axis_index("c")
        base = wid * ROWS_PER_SC
        iota = lax.iota(jnp.int32, 16)
        fzero16 = jnp.zeros((16,), jnp.float32)
        izero16 = jnp.zeros((16,), jnp.int32)
        BIG = jnp.int32(0x7FFFFFF)

        def _zero_row(r, _):
            for cc in range(CAP // 16):
                Vt_v[r, pl.ds(cc * 16, 16)] = fzero16
                Dt_v[r, pl.ds(cc * 16, 16)] = izero16
            return 0
        lax.fori_loop(0, ROWS_PER_SC, _zero_row, 0)

        def _zero_cnt(i, _):
            rcnt_v[pl.ds(i * 16, 16)] = izero16
            return 0
        lax.fori_loop(0, ROWS_PER_SC // 16, _zero_cnt, 0)

        def bucket_body(u, _):
            pltpu.async_copy(bs_hbm.at[u, wid], eS_v, sem).wait()
            pltpu.async_copy(bd_hbm.at[u, wid], eD_v, sem).wait()
            pltpu.async_copy(bv_hbm.at[u, wid], eV_v, sem).wait()

            def vreg_body(g, _):
                s16 = eS_v[pl.ds(g * 16, 16)]
                valid = s16 < SENT
                key = jnp.where(valid, s16 - base, BIG)
                ks, perm, p, last = _vreg_group_slots(key, iota, tmp_v)
                vs = ks != BIG
                ksc = jnp.clip(ks, 0, ROWS_PER_SC - 1)
                cold = plsc.load_gather(rcnt_v, [ksc])
                slot = cold + p
                ok = vs & (slot < CAP)
                plsc.store_scatter(rcnt_v, [ksc], jnp.minimum(slot + 1, CAP),
                                   mask=last & vs)
                pabs = perm + g * 16
                dp = plsc.load_gather(eD_v, [pabs])
                vp = plsc.load_gather(eV_v, [pabs])
                plsc.store_scatter(Dt_v, [ksc, slot], dp, mask=ok)
                plsc.store_scatter(Vt_v, [ksc, slot], vp, mask=ok)
                return 0
            lax.fori_loop(0, BCAP // 16, vreg_body, 0)
            return 0

        lax.fori_loop(0, NW, bucket_body, 0)
        pltpu.sync_copy(Vt_v, vt_hbm.at[pl.ds(base, ROWS_PER_SC)])
        pltpu.sync_copy(Dt_v, dt_hbm.at[pl.ds(base, ROWS_PER_SC)])

    return kern(bS, bD, bV)


def _rowproc_kernel(v_ref, d_ref, w3_ref, b3_ref, out_ref, c_ref, p_ref, r_ref):
    Vt = v_ref[...]
    Dt = d_ref[...]
    w3 = w3_ref[0, 0]
    b3 = b3_ref[0, 0]
    rowsum = jnp.sum(Vt, axis=0, keepdims=True)
    k = _leaky(rowsum * w3 + b3)
    aidx = lax.broadcasted_iota(jnp.int32, (CAP, 1), 0)

    for a in range(CAP):
        Da = Dt[a : a + 1, :]
        eq = Dt == Da
        c_ref[a : a + 1, :] = jnp.sum(jnp.where(eq, Vt, 0.0), axis=0, keepdims=True)
        seen = jnp.sum((eq & (aidx < a)).astype(jnp.int32), axis=0, keepdims=True)
        p_ref[a : a + 1, :] = (seen == 0).astype(jnp.int32)

    Ct = c_ref[...]
    Pt = p_ref[...] != 0
    for a in range(CAP):
        Da = Dt[a : a + 1, :]
        Ca = Ct[a : a + 1, :]
        beats = (Ct > Ca) | ((Ct == Ca) & (Dt < Da))
        r_ref[a : a + 1, :] = jnp.sum(
            jnp.where(beats & Pt, 1.0, 0.0), axis=0, keepdims=True)

    f = 1.5 - 0.5 * jnp.tanh(r_ref[...] - k)
    out_ref[...] = jnp.where(Pt, Ct * f, 0.0)


def _rowproc(VtabT, DtabT, w3, b3):
    n_pad = VtabT.shape[1]
    blk = 512
    return pl.pallas_call(
        _rowproc_kernel,
        grid=(n_pad // blk,),
        in_specs=[
            pl.BlockSpec((CAP, blk), lambda i: (0, i)),
            pl.BlockSpec((CAP, blk), lambda i: (0, i)),
            pl.BlockSpec((1, 1), lambda i: (0, 0)),
            pl.BlockSpec((1, 1), lambda i: (0, 0)),
        ],
        out_specs=pl.BlockSpec((CAP, blk), lambda i: (0, i)),
        out_shape=jax.ShapeDtypeStruct((CAP, n_pad), jnp.float32),
        scratch_shapes=[
            pltpu.VMEM((CAP, blk), jnp.float32),
            pltpu.VMEM((CAP, blk), jnp.int32),
            pltpu.VMEM((CAP, blk), jnp.float32),
        ],
    )(VtabT, DtabT, w3, b3.reshape(1, 1))


def _materialize(out_val, Dtab, N):
    G = CAP // 16

    mesh = plsc.VectorSubcoreMesh(core_axis_name="c", subcore_axis_name="s")

    @functools.partial(
        pl.kernel,
        out_type=jax.ShapeDtypeStruct((N, N), jnp.float32),
        mesh=mesh,
        compiler_params=_SC_PARAMS,
        scratch_types=[
            pltpu.VMEM((ROW_BLK, CAP), jnp.float32),
            pltpu.VMEM((ROW_BLK, CAP), jnp.int32),
            pltpu.VMEM((N,), jnp.float32),
            pltpu.VMEM((N,), jnp.float32),
            pltpu.VMEM((2, CAP), jnp.int32),
            pltpu.SemaphoreType.DMA,
            pltpu.SemaphoreType.DMA,
            pltpu.SemaphoreType.DMA,
        ],
    )
    def kern(val_hbm, col_hbm, out_hbm, val_v, col_v, buf0, buf1, pend, sem0, sem1, sem_in):
        wid = lax.axis_index("s") * 2 + lax.axis_index("c")
        base = wid * ROWS_PER_SC
        nrows = jnp.minimum(ROWS_PER_SC, N - base)
        nblk = nrows // ROW_BLK
        zeros16 = jnp.zeros((16,), jnp.float32)

        def _zero_init(i, _):
            buf0[pl.ds(i * 16, 16)] = zeros16
            buf1[pl.ds(i * 16, 16)] = zeros16
            return 0
        lax.fori_loop(0, N // 16, _zero_init, 0)

        def blk_body(b, first_flags):
            f0, f1 = first_flags
            pltpu.async_copy(
                val_hbm.at[pl.ds(base + b * ROW_BLK, ROW_BLK)], val_v, sem_in
            ).wait()
            pltpu.async_copy(
                col_hbm.at[pl.ds(base + b * ROW_BLK, ROW_BLK)], col_v, sem_in
            ).wait()

            def pair_body(p, flags):
                ff0, ff1 = flags

                def one(local_r, buf, sem, pslot, first):
                    @pl.when(jnp.logical_not(first))
                    def _():
                        pltpu.make_async_copy(buf, out_hbm.at[0], sem).wait()
                        for g in range(G):
                            idx = pend[pslot, pl.ds(g * 16, 16)]
                            plsc.store_scatter(buf, [idx], zeros16)
                    for g in range(G):
                        cols = col_v[local_r, pl.ds(g * 16, 16)]
                        vals = val_v[local_r, pl.ds(g * 16, 16)]
                        plsc.addupdate_scatter(buf, [cols], vals)
                        pend[pslot, pl.ds(g * 16, 16)] = cols
                    row = base + b * ROW_BLK + local_r
                    pltpu.async_copy(buf, out_hbm.at[row], sem)

                one(2 * p, buf0, sem0, 0, ff0)
                one(2 * p + 1, buf1, sem1, 1, ff1)
                return (jnp.bool_(False), jnp.bool_(False))

            return lax.fori_loop(0, ROW_BLK // 2, pair_body, (f0, f1))

        flags = lax.fori_loop(0, nblk, blk_body, (jnp.bool_(True), jnp.bool_(True)))
        @pl.when(jnp.logical_not(flags[0]))
        def _():
            pltpu.make_async_copy(buf0, out_hbm.at[0], sem0).wait()
        @pl.when(jnp.logical_not(flags[1]))
        def _():
            pltpu.make_async_copy(buf1, out_hbm.at[0], sem1).wait()

    return kern(out_val, Dtab)


def kernel(x, edge_index, W1, b1, W2, b2, w3, b3):
    src = edge_index[0].astype(jnp.int32)
    dst = edge_index[1].astype(jnp.int32)
    x_enc, y, yd = _encode(x, W1, b1, W2, b2)
    bS, bD, bV = _edge_bucket(y, yd, src, dst)
    n_pad = NW * ROWS_PER_SC
    Vtab, Dtab = _build_tables(bS, bD, bV, n_pad)
    out_valT = _rowproc(Vtab.T, Dtab.T, w3, b3)
    out = _materialize(out_valT.T, Dtab, x.shape[0])
    return out, x_enc


# speedup vs baseline: 739.2766x; 1.0441x over previous
import functools

import jax
import jax.numpy as jnp
from jax import lax
from jax.experimental import pallas as pl
from jax.experimental.pallas import tpu as pltpu
from jax.experimental.pallas import tpu_sc as plsc

CAP = 80
ROWS_PER_SC = 320
ROW_BLK = 80
NW = 32

_SC_PARAMS = pltpu.CompilerParams(needs_layout_passes=False)


def _leaky(t):
    return jnp.maximum(t, 0.01 * t)


def _encode_kernel(x_ref, w1_ref, b1_ref, w2_ref, b2_ref, xe_ref, y_ref, yd_ref):
    h = jnp.dot(x_ref[...], w1_ref[...], preferred_element_type=jnp.float32)
    h = h + b1_ref[...]
    xe = _leaky(h)
    xe_ref[...] = xe
    y = jnp.dot(xe, w2_ref[...], preferred_element_type=jnp.float32)
    y_ref[...] = y
    yd_ref[...] = y - b2_ref[...]


def _encode(x, W1, b1, W2, b2):
    N, d_in = x.shape
    d_lat = W1.shape[1]
    blk = 400
    grid = (N // blk,)
    return pl.pallas_call(
        _encode_kernel,
        grid=grid,
        in_specs=[
            pl.BlockSpec((blk, d_in), lambda i: (i, 0)),
            pl.BlockSpec((d_in, d_lat), lambda i: (0, 0)),
            pl.BlockSpec((1, d_lat), lambda i: (0, 0)),
            pl.BlockSpec((d_lat, d_lat), lambda i: (0, 0)),
            pl.BlockSpec((1, d_lat), lambda i: (0, 0)),
        ],
        out_specs=[
            pl.BlockSpec((blk, d_lat), lambda i: (i, 0)),
            pl.BlockSpec((blk, d_lat), lambda i: (i, 0)),
            pl.BlockSpec((blk, d_lat), lambda i: (i, 0)),
        ],
        out_shape=[
            jax.ShapeDtypeStruct((N, d_lat), jnp.float32),
            jax.ShapeDtypeStruct((N, d_lat), jnp.float32),
            jax.ShapeDtypeStruct((N, d_lat), jnp.float32),
        ],
    )(x, W1, b1.reshape(1, -1), W2, b2.reshape(1, -1))


BCAP = 512
SENT = 0x7FFF0000
CHUNK = 80


def _vreg_group_slots(key, iota, tmp_v):
    ks, perm = plsc.sort_key_val(key, iota)
    tmp_v[...] = ks
    prv = plsc.load_gather(tmp_v, [jnp.maximum(iota - 1, 0)])
    nxt = plsc.load_gather(tmp_v, [jnp.minimum(iota + 1, 15)])
    new = (iota == 0) | (ks != prv)
    m0 = plsc.cummax(jnp.where(new, iota, 0))
    p = iota - m0
    last = (iota == 15) | (ks != nxt)
    return ks, perm, p, last


def _edge_bucket(y, yd, src, dst):
    N, d_lat = y.shape
    E = src.shape[0]
    epw = E // NW
    nchunks = epw // CHUNK

    mesh = plsc.VectorSubcoreMesh(core_axis_name="c", subcore_axis_name="s")

    @functools.partial(
        pl.kernel,
        out_type=(jax.ShapeDtypeStruct((NW, NW, BCAP), jnp.int32),
                  jax.ShapeDtypeStruct((NW, NW, BCAP), jnp.int32),
                  jax.ShapeDtypeStruct((NW, NW, BCAP), jnp.float32)),
        mesh=mesh,
        compiler_params=_SC_PARAMS,
        scratch_types=[
            pltpu.VMEM((CHUNK,), jnp.int32),
            pltpu.VMEM((CHUNK,), jnp.int32),
            pltpu.VMEM((CHUNK, 128), jnp.float32),
            pltpu.VMEM((CHUNK, 128), jnp.float32),
            pltpu.VMEM((CHUNK,), jnp.int32),
            pltpu.VMEM((CHUNK,), jnp.int32),
            pltpu.VMEM((CHUNK, 128), jnp.float32),
            pltpu.VMEM((CHUNK, 128), jnp.float32),
            pltpu.VMEM((CHUNK,), jnp.float32),
            pltpu.VMEM((16 * 17,), jnp.float32),
            pltpu.VMEM((16,), jnp.float32),
            pltpu.VMEM((NW, BCAP), jnp.int32),
            pltpu.VMEM((NW, BCAP), jnp.int32),
            pltpu.VMEM((NW, BCAP), jnp.float32),
            pltpu.VMEM((NW,), jnp.int32),
            pltpu.VMEM((16,), jnp.int32),
            pltpu.SemaphoreType.DMA,
            pltpu.SemaphoreType.DMA,
            pltpu.SemaphoreType.DMA,
            pltpu.SemaphoreType.DMA,
        ],
    )
    def kern(src_hbm, dst_hbm, y_hbm, yd_hbm, bs_hbm, bd_hbm, bv_hbm,
             src_v0, dst_v0, ysrc_v0, ydst_v0, src_v1, dst_v1, ysrc_v1, ydst_v1,
             sbuf, accT, sb16, bS_v, bD_v, bV_v, cnt_v,
             tmp_v, sem_a, sem_b, sem_g0, sem_g1):
        wid = lax.axis_index("s") * 2 + lax.axis_index("c")
        ebase = wid * epw
        iota = lax.iota(jnp.int32, 16)
        sent16 = jnp.full((16,), SENT, jnp.int32)
        izero16 = jnp.zeros((16,), jnp.int32)

        def _prefill(i, _):
            r = i // (BCAP // 16)
            c = i % (BCAP // 16)
            bS_v[r, pl.ds(c * 16, 16)] = sent16
            return 0
        lax.fori_loop(0, NW * (BCAP // 16), _prefill, 0)
        cnt_v[pl.ds(0, 16)] = izero16
        cnt_v[pl.ds(16, 16)] = izero16

        bufs = ((src_v0, dst_v0, ysrc_v0, ydst_v0, sem_g0),
                (src_v1, dst_v1, ysrc_v1, ydst_v1, sem_g1))

        def issue(c, p):
            src_v, dst_v, ysrc_v, ydst_v, sem_g = bufs[p]
            off = ebase + c * CHUNK
            h1 = pltpu.async_copy(src_hbm.at[pl.ds(off, CHUNK)], src_v, sem_a)
            h2 = pltpu.async_copy(dst_hbm.at[pl.ds(off, CHUNK)], dst_v, sem_b)
            h1.wait()
            h2.wait()
            pltpu.async_copy(y_hbm.at[src_v], ysrc_v, sem_g)
            pltpu.async_copy(yd_hbm.at[dst_v], ydst_v, sem_g)

        def compute(p):
            src_v, dst_v, ysrc_v, ydst_v, sem_g = bufs[p]
            pltpu.make_async_copy(y_hbm.at[src_v], ysrc_v, sem_g).wait()
            pltpu.make_async_copy(yd_hbm.at[dst_v], ydst_v, sem_g).wait()

            def batch16(eb, _):
                for j in range(16):
                    e = eb * 16 + j
                    acc0 = jnp.zeros((16,), jnp.float32)
                    acc1 = jnp.zeros((16,), jnp.float32)
                    for cc in range(d_lat // 32):
                        a = ysrc_v[e, pl.ds(cc * 16, 16)]
                        b = ydst_v[e, pl.ds(cc * 16, 16)]
                        t = a - b
                        acc0 = acc0 + jnp.maximum(t, 0.01 * t)
                    for cc in range(d_lat // 32, d_lat // 16):
                        a = ysrc_v[e, pl.ds(cc * 16, 16)]
                        b = ydst_v[e, pl.ds(cc * 16, 16)]
                        t = a - b
                        acc1 = acc1 + jnp.maximum(t, 0.01 * t)
                    plsc.store_scatter(accT, [iota * 17 + j], acc0 + acc1)
                parts = [accT[pl.ds(l * 17, 16)] for l in range(16)]
                while len(parts) > 1:
                    parts = [parts[i] + parts[i + 1] for i in range(0, len(parts), 2)]
                v16 = 1.0 / (1.0 + jnp.exp(-parts[0]))
                sb16[...] = v16
                s16 = src_v[pl.ds(eb * 16, 16)]
                b16 = s16 // ROWS_PER_SC
                ks, perm, p_, last = _vreg_group_slots(b16, iota, tmp_v)
                cold = plsc.load_gather(cnt_v, [ks])
                slot = cold + p_
                ok = slot < BCAP
                plsc.store_scatter(cnt_v, [ks], jnp.minimum(slot + 1, BCAP),
                                   mask=last)
                pabs = perm + eb * 16
                sp = plsc.load_gather(src_v, [pabs])
                dp = plsc.load_gather(dst_v, [pabs])
                vp = plsc.load_gather(sb16, [perm])
                plsc.store_scatter(bS_v, [ks, slot], sp, mask=ok)
                plsc.store_scatter(bD_v, [ks, slot], dp, mask=ok)
                plsc.store_scatter(bV_v, [ks, slot], vp, mask=ok)
                return 0
            lax.fori_loop(0, CHUNK // 16, batch16, 0)

        issue(0, 0)

        def pair_body(i, _):
            issue(2 * i + 1, 1)
            compute(0)
            issue(2 * i + 2, 0)
            compute(1)
            return 0
        lax.fori_loop(0, (nchunks - 1) // 2, pair_body, 0)
        compute(0)

        pltpu.sync_copy(bS_v, bs_hbm.at[wid])
        pltpu.sync_copy(bD_v, bd_hbm.at[wid])
        pltpu.sync_copy(bV_v, bv_hbm.at[wid])

    return kern(src, dst, y, yd)


def _build_tables(bS, bD, bV, n_pad):
    mesh = plsc.VectorSubcoreMesh(core_axis_name="c", subcore_axis_name="s")

    @functools.partial(
        pl.kernel,
        out_type=(jax.ShapeDtypeStruct((n_pad, CAP), jnp.float32),
                  jax.ShapeDtypeStruct((n_pad, CAP), jnp.int32)),
        mesh=mesh,
        compiler_params=_SC_PARAMS,
        scratch_types=[
            pltpu.VMEM((ROWS_PER_SC, CAP), jnp.float32),
            pltpu.VMEM((ROWS_PER_SC, CAP), jnp.int32),
            pltpu.VMEM((ROWS_PER_SC,), jnp.int32),
            pltpu.VMEM((BCAP,), jnp.int32),
            pltpu.VMEM((BCAP,), jnp.int32),
            pltpu.VMEM((BCAP,), jnp.float32),
            pltpu.VMEM((16,), jnp.int32),
            pltpu.SemaphoreType.DMA,
        ],
    )
    def kern(bs_hbm, bd_hbm, bv_hbm, vt_hbm, dt_hbm,
             Vt_v, Dt_v, rcnt_v, eS_v, eD_v, eV_v, tmp_v, sem):
        wid = lax.axis_index("s") * 2 + lax.axis_index("c")
        base = wid * ROWS_PER_SC
        iota = lax.iota(jnp.int32, 16)
        fzero16 = jnp.zeros((16,), jnp.float32)
        izero16 = jnp.zeros((16,), jnp.int32)
        BIG = jnp.int32(0x7FFFFFF)

        def _zero_row(r, _):
            for cc in range(CAP // 16):
                Vt_v[r, pl.ds(cc * 16, 16)] = fzero16
                Dt_v[r, pl.ds(cc * 16, 16)] = izero16
            return 0
        lax.fori_loop(0, ROWS_PER_SC, _zero_row, 0)

        def _zero_cnt(i, _):
            rcnt_v[pl.ds(i * 16, 16)] = izero16
            return 0
        lax.fori_loop(0, ROWS_PER_SC // 16, _zero_cnt, 0)

        def bucket_body(u, _):
            pltpu.async_copy(bs_hbm.at[u, wid], eS_v, sem).wait()
            pltpu.async_copy(bd_hbm.at[u, wid], eD_v, sem).wait()
            pltpu.async_copy(bv_hbm.at[u, wid], eV_v, sem).wait()

            def vreg_body(g, _):
                s16 = eS_v[pl.ds(g * 16, 16)]
                valid = s16 < SENT
                key = jnp.where(valid, s16 - base, BIG)
                ks, perm, p, last = _vreg_group_slots(key, iota, tmp_v)
                vs = ks != BIG
                ksc = jnp.clip(ks, 0, ROWS_PER_SC - 1)
                cold = plsc.load_gather(rcnt_v, [ksc])
                slot = cold + p
                ok = vs & (slot < CAP)
                plsc.store_scatter(rcnt_v, [ksc], jnp.minimum(slot + 1, CAP),
                                   mask=last & vs)
                pabs = perm + g * 16
                dp = plsc.load_gather(eD_v, [pabs])
                vp = plsc.load_gather(eV_v, [pabs])
                plsc.store_scatter(Dt_v, [ksc, slot], dp, mask=ok)
                plsc.store_scatter(Vt_v, [ksc, slot], vp, mask=ok)
                return 0
            lax.fori_loop(0, BCAP // 16, vreg_body, 0)
            return 0

        lax.fori_loop(0, NW, bucket_body, 0)
        pltpu.sync_copy(Vt_v, vt_hbm.at[pl.ds(base, ROWS_PER_SC)])
        pltpu.sync_copy(Dt_v, dt_hbm.at[pl.ds(base, ROWS_PER_SC)])

    return kern(bS, bD, bV)


def _rowproc_kernel(v_ref, d_ref, w3_ref, b3_ref, out_ref, c_ref, p_ref, r_ref):
    Vt = v_ref[...]
    Dt = d_ref[...]
    w3 = w3_ref[0, 0]
    b3 = b3_ref[0, 0]
    rowsum = jnp.sum(Vt, axis=0, keepdims=True)
    k = _leaky(rowsum * w3 + b3)
    aidx = lax.broadcasted_iota(jnp.int32, (CAP, 1), 0)

    for a in range(CAP):
        Da = Dt[a : a + 1, :]
        eq = Dt == Da
        c_ref[a : a + 1, :] = jnp.sum(jnp.where(eq, Vt, 0.0), axis=0, keepdims=True)
        seen = jnp.sum((eq & (aidx < a)).astype(jnp.int32), axis=0, keepdims=True)
        p_ref[a : a + 1, :] = (seen == 0).astype(jnp.int32)

    Ct = c_ref[...]
    Pt = p_ref[...] != 0
    for a in range(CAP):
        Da = Dt[a : a + 1, :]
        Ca = Ct[a : a + 1, :]
        beats = (Ct > Ca) | ((Ct == Ca) & (Dt < Da))
        r_ref[a : a + 1, :] = jnp.sum(
            jnp.where(beats & Pt, 1.0, 0.0), axis=0, keepdims=True)

    f = 1.5 - 0.5 * jnp.tanh(r_ref[...] - k)
    out_ref[...] = jnp.where(Pt, Ct * f, 0.0)


def _rowproc(VtabT, DtabT, w3, b3):
    n_pad = VtabT.shape[1]
    blk = 512
    return pl.pallas_call(
        _rowproc_kernel,
        grid=(n_pad // blk,),
        in_specs=[
            pl.BlockSpec((CAP, blk), lambda i: (0, i)),
            pl.BlockSpec((CAP, blk), lambda i: (0, i)),
            pl.BlockSpec((1, 1), lambda i: (0, 0)),
            pl.BlockSpec((1, 1), lambda i: (0, 0)),
        ],
        out_specs=pl.BlockSpec((CAP, blk), lambda i: (0, i)),
        out_shape=jax.ShapeDtypeStruct((CAP, n_pad), jnp.float32),
        scratch_shapes=[
            pltpu.VMEM((CAP, blk), jnp.float32),
            pltpu.VMEM((CAP, blk), jnp.int32),
            pltpu.VMEM((CAP, blk), jnp.float32),
        ],
    )(VtabT, DtabT, w3, b3.reshape(1, 1))


def _materialize(out_val, Dtab, N):
    G = CAP // 16

    mesh = plsc.VectorSubcoreMesh(core_axis_name="c", subcore_axis_name="s")

    @functools.partial(
        pl.kernel,
        out_type=jax.ShapeDtypeStruct((N, N), jnp.float32),
        mesh=mesh,
        compiler_params=_SC_PARAMS,
        scratch_types=[
            pltpu.VMEM((ROW_BLK, CAP), jnp.float32),
            pltpu.VMEM((ROW_BLK, CAP), jnp.int32),
            pltpu.VMEM((N,), jnp.float32),
            pltpu.VMEM((N,), jnp.float32),
            pltpu.VMEM((2, CAP), jnp.int32),
            pltpu.SemaphoreType.DMA,
            pltpu.SemaphoreType.DMA,
            pltpu.SemaphoreType.DMA,
        ],
    )
    def kern(val_hbm, col_hbm, out_hbm, val_v, col_v, buf0, buf1, pend, sem0, sem1, sem_in):
        wid = lax.axis_index("s") * 2 + lax.axis_index("c")
        base = wid * ROWS_PER_SC
        nrows = jnp.minimum(ROWS_PER_SC, N - base)
        nblk = nrows // ROW_BLK
        zeros16 = jnp.zeros((16,), jnp.float32)

        def _zero_init(i, _):
            buf0[pl.ds(i * 16, 16)] = zeros16
            buf1[pl.ds(i * 16, 16)] = zeros16
            return 0
        lax.fori_loop(0, N // 16, _zero_init, 0)

        def blk_body(b, first_flags):
            f0, f1 = first_flags
            pltpu.async_copy(
                val_hbm.at[pl.ds(base + b * ROW_BLK, ROW_BLK)], val_v, sem_in
            ).wait()
            pltpu.async_copy(
                col_hbm.at[pl.ds(base + b * ROW_BLK, ROW_BLK)], col_v, sem_in
            ).wait()

            def pair_body(p, flags):
                ff0, ff1 = flags

                def one(local_r, buf, sem, pslot, first):
                    @pl.when(jnp.logical_not(first))
                    def _():
                        pltpu.make_async_copy(buf, out_hbm.at[0], sem).wait()
                        for g in range(G):
                            idx = pend[pslot, pl.ds(g * 16, 16)]
                            plsc.store_scatter(buf, [idx], zeros16)
                    for g in range(G):
                        cols = col_v[local_r, pl.ds(g * 16, 16)]
                        vals = val_v[local_r, pl.ds(g * 16, 16)]
                        plsc.addupdate_scatter(buf, [cols], vals)
                        pend[pslot, pl.ds(g * 16, 16)] = cols
                    row = base + b * ROW_BLK + local_r
                    pltpu.async_copy(buf, out_hbm.at[row], sem)

                one(2 * p, buf0, sem0, 0, ff0)
                one(2 * p + 1, buf1, sem1, 1, ff1)
                return (jnp.bool_(False), jnp.bool_(False))

            return lax.fori_loop(0, ROW_BLK // 2, pair_body, (f0, f1))

        flags = lax.fori_loop(0, nblk, blk_body, (jnp.bool_(True), jnp.bool_(True)))
        @pl.when(jnp.logical_not(flags[0]))
        def _():
            pltpu.make_async_copy(buf0, out_hbm.at[0], sem0).wait()
        @pl.when(jnp.logical_not(flags[1]))
        def _():
            pltpu.make_async_copy(buf1, out_hbm.at[0], sem1).wait()

    return kern(out_val, Dtab)


def kernel(x, edge_index, W1, b1, W2, b2, w3, b3):
    src = edge_index[0].astype(jnp.int32)
    dst = edge_index[1].astype(jnp.int32)
    x_enc, y, yd = _encode(x, W1, b1, W2, b2)
    bS, bD, bV = _edge_bucket(y, yd, src, dst)
    n_pad = NW * ROWS_PER_SC
    Vtab, Dtab = _build_tables(bS, bD, bV, n_pad)
    out_valT = _rowproc(Vtab.T, Dtab.T, w3, b3)
    out = _materialize(out_valT.T, Dtab, x.shape[0])
    return out, x_enc


# speedup vs baseline: 777.6698x; 1.0519x over previous
import functools

import jax
import jax.numpy as jnp
from jax import lax
from jax.experimental import pallas as pl
from jax.experimental.pallas import tpu as pltpu
from jax.experimental.pallas import tpu_sc as plsc

CAP = 80
ROWS_PER_SC = 320
ROW_BLK = 80
NW = 32

_SC_PARAMS = pltpu.CompilerParams(needs_layout_passes=False)


def _leaky(t):
    return jnp.maximum(t, 0.01 * t)


def _encode_kernel(x_ref, w1_ref, b1_ref, w2_ref, b2_ref, xe_ref, y_ref, yd_ref):
    h = jnp.dot(x_ref[...], w1_ref[...], preferred_element_type=jnp.float32)
    h = h + b1_ref[...]
    xe = _leaky(h)
    xe_ref[...] = xe
    y = jnp.dot(xe, w2_ref[...], preferred_element_type=jnp.float32)
    y_ref[...] = y
    yd_ref[...] = y - b2_ref[...]


def _encode(x, W1, b1, W2, b2):
    N, d_in = x.shape
    d_lat = W1.shape[1]
    blk = 400
    grid = (N // blk,)
    return pl.pallas_call(
        _encode_kernel,
        grid=grid,
        in_specs=[
            pl.BlockSpec((blk, d_in), lambda i: (i, 0)),
            pl.BlockSpec((d_in, d_lat), lambda i: (0, 0)),
            pl.BlockSpec((1, d_lat), lambda i: (0, 0)),
            pl.BlockSpec((d_lat, d_lat), lambda i: (0, 0)),
            pl.BlockSpec((1, d_lat), lambda i: (0, 0)),
        ],
        out_specs=[
            pl.BlockSpec((blk, d_lat), lambda i: (i, 0)),
            pl.BlockSpec((blk, d_lat), lambda i: (i, 0)),
            pl.BlockSpec((blk, d_lat), lambda i: (i, 0)),
        ],
        out_shape=[
            jax.ShapeDtypeStruct((N, d_lat), jnp.float32),
            jax.ShapeDtypeStruct((N, d_lat), jnp.float32),
            jax.ShapeDtypeStruct((N, d_lat), jnp.float32),
        ],
    )(x, W1, b1.reshape(1, -1), W2, b2.reshape(1, -1))


BCAP = 512
SENT = 0x7FFF0000
CHUNK = 80


def _vreg_group_slots(key, iota, tmp_v):
    ks, perm = plsc.sort_key_val(key, iota)
    tmp_v[...] = ks
    prv = plsc.load_gather(tmp_v, [jnp.maximum(iota - 1, 0)])
    nxt = plsc.load_gather(tmp_v, [jnp.minimum(iota + 1, 15)])
    new = (iota == 0) | (ks != prv)
    m0 = plsc.cummax(jnp.where(new, iota, 0))
    p = iota - m0
    last = (iota == 15) | (ks != nxt)
    return ks, perm, p, last


def _edge_bucket(y, yd, src, dst):
    N, d_lat = y.shape
    E = src.shape[0]
    epw = E // NW
    nchunks = epw // CHUNK

    mesh = plsc.VectorSubcoreMesh(core_axis_name="c", subcore_axis_name="s")

    @functools.partial(
        pl.kernel,
        out_type=(jax.ShapeDtypeStruct((NW, NW, BCAP), jnp.int32),
                  jax.ShapeDtypeStruct((NW, NW, BCAP), jnp.int32),
                  jax.ShapeDtypeStruct((NW, NW, BCAP), jnp.float32)),
        mesh=mesh,
        compiler_params=_SC_PARAMS,
        scratch_types=[
            pltpu.VMEM((CHUNK,), jnp.int32),
            pltpu.VMEM((CHUNK,), jnp.int32),
            pltpu.VMEM((CHUNK, 128), jnp.float32),
            pltpu.VMEM((CHUNK, 128), jnp.float32),
            pltpu.VMEM((CHUNK,), jnp.int32),
            pltpu.VMEM((CHUNK,), jnp.int32),
            pltpu.VMEM((CHUNK, 128), jnp.float32),
            pltpu.VMEM((CHUNK, 128), jnp.float32),
            pltpu.VMEM((CHUNK,), jnp.float32),
            pltpu.VMEM((16 * 17,), jnp.float32),
            pltpu.VMEM((16,), jnp.float32),
            pltpu.VMEM((NW, BCAP), jnp.int32),
            pltpu.VMEM((NW, BCAP), jnp.int32),
            pltpu.VMEM((NW, BCAP), jnp.float32),
            pltpu.VMEM((NW,), jnp.int32),
            pltpu.VMEM((16,), jnp.int32),
            pltpu.SemaphoreType.DMA,
            pltpu.SemaphoreType.DMA,
            pltpu.SemaphoreType.DMA,
            pltpu.SemaphoreType.DMA,
        ],
    )
    def kern(src_hbm, dst_hbm, y_hbm, yd_hbm, bs_hbm, bd_hbm, bv_hbm,
             src_v0, dst_v0, ysrc_v0, ydst_v0, src_v1, dst_v1, ysrc_v1, ydst_v1,
             sbuf, accT, sb16, bS_v, bD_v, bV_v, cnt_v,
             tmp_v, sem_a, sem_b, sem_g0, sem_g1):
        wid = lax.axis_index("s") * 2 + lax.axis_index("c")
        ebase = wid * epw
        iota = lax.iota(jnp.int32, 16)
        sent16 = jnp.full((16,), SENT, jnp.int32)
        izero16 = jnp.zeros((16,), jnp.int32)

        def _prefill(i, _):
            r = i // (BCAP // 16)
            c = i % (BCAP // 16)
            bS_v[r, pl.ds(c * 16, 16)] = sent16
            return 0
        lax.fori_loop(0, NW * (BCAP // 16), _prefill, 0)
        cnt_v[pl.ds(0, 16)] = izero16
        cnt_v[pl.ds(16, 16)] = izero16

        bufs = ((src_v0, dst_v0, ysrc_v0, ydst_v0, sem_g0),
                (src_v1, dst_v1, ysrc_v1, ydst_v1, sem_g1))

        def issue(c, p):
            src_v, dst_v, ysrc_v, ydst_v, sem_g = bufs[p]
            off = ebase + c * CHUNK
            h1 = pltpu.async_copy(src_hbm.at[pl.ds(off, CHUNK)], src_v, sem_a)
            h2 = pltpu.async_copy(dst_hbm.at[pl.ds(off, CHUNK)], dst_v, sem_b)
            h1.wait()
            h2.wait()
            pltpu.async_copy(y_hbm.at[src_v], ysrc_v, sem_g)
            pltpu.async_copy(yd_hbm.at[dst_v], ydst_v, sem_g)

        def compute(p):
            src_v, dst_v, ysrc_v, ydst_v, sem_g = bufs[p]
            pltpu.make_async_copy(y_hbm.at[src_v], ysrc_v, sem_g).wait()
            pltpu.make_async_copy(yd_hbm.at[dst_v], ydst_v, sem_g).wait()

            def batch16(eb, _):
                for j in range(16):
                    e = eb * 16 + j
                    acc0 = jnp.zeros((16,), jnp.float32)
                    acc1 = jnp.zeros((16,), jnp.float32)
                    for cc in range(d_lat // 32):
                        a = ysrc_v[e, pl.ds(cc * 16, 16)]
                        b = ydst_v[e, pl.ds(cc * 16, 16)]
                        t = a - b
                        acc0 = acc0 + jnp.maximum(t, 0.01 * t)
                    for cc in range(d_lat // 32, d_lat // 16):
                        a = ysrc_v[e, pl.ds(cc * 16, 16)]
                        b = ydst_v[e, pl.ds(cc * 16, 16)]
                        t = a - b
                        acc1 = acc1 + jnp.maximum(t, 0.01 * t)
                    plsc.store_scatter(accT, [iota * 17 + j], acc0 + acc1)
                parts = [accT[pl.ds(l * 17, 16)] for l in range(16)]
                while len(parts) > 1:
                    parts = [parts[i] + parts[i + 1] for i in range(0, len(parts), 2)]
                v16 = 1.0 / (1.0 + jnp.exp(-parts[0]))
                sb16[...] = v16
                s16 = src_v[pl.ds(eb * 16, 16)]
                b16 = s16 // ROWS_PER_SC
                ks, perm, p_, last = _vreg_group_slots(b16, iota, tmp_v)
                cold = plsc.load_gather(cnt_v, [ks])
                slot = cold + p_
                ok = slot < BCAP
                plsc.store_scatter(cnt_v, [ks], jnp.minimum(slot + 1, BCAP),
                                   mask=last)
                pabs = perm + eb * 16
                sp = plsc.load_gather(src_v, [pabs])
                dp = plsc.load_gather(dst_v, [pabs])
                vp = plsc.load_gather(sb16, [perm])
                plsc.store_scatter(bS_v, [ks, slot], sp, mask=ok)
                plsc.store_scatter(bD_v, [ks, slot], dp, mask=ok)
                plsc.store_scatter(bV_v, [ks, slot], vp, mask=ok)
                return 0
            lax.fori_loop(0, CHUNK // 16, batch16, 0)

        issue(0, 0)

        def pair_body(i, _):
            issue(2 * i + 1, 1)
            compute(0)
            issue(2 * i + 2, 0)
            compute(1)
            return 0
        lax.fori_loop(0, (nchunks - 1) // 2, pair_body, 0)
        compute(0)

        pltpu.sync_copy(bS_v, bs_hbm.at[wid])
        pltpu.sync_copy(bD_v, bd_hbm.at[wid])
        pltpu.sync_copy(bV_v, bv_hbm.at[wid])

    return kern(src, dst, y, yd)


def _build_tables(bS, bD, bV, n_pad):
    mesh = plsc.VectorSubcoreMesh(core_axis_name="c", subcore_axis_name="s")

    @functools.partial(
        pl.kernel,
        out_type=(jax.ShapeDtypeStruct((n_pad * CAP,), jnp.float32),
                  jax.ShapeDtypeStruct((n_pad * CAP,), jnp.int32)),
        mesh=mesh,
        compiler_params=_SC_PARAMS,
        scratch_types=[
            pltpu.VMEM((ROWS_PER_SC * CAP,), jnp.float32),
            pltpu.VMEM((ROWS_PER_SC * CAP,), jnp.int32),
            pltpu.VMEM((ROWS_PER_SC,), jnp.int32),
            pltpu.VMEM((NW, BCAP), jnp.int32),
            pltpu.VMEM((NW, BCAP), jnp.int32),
            pltpu.VMEM((NW, BCAP), jnp.float32),
            pltpu.VMEM((16,), jnp.int32),
            pltpu.SemaphoreType.DMA,
        ],
    )
    def kern(bs_hbm, bd_hbm, bv_hbm, vt_hbm, dt_hbm,
             Vt_v, Dt_v, rcnt_v, eS_v, eD_v, eV_v, tmp_v, sem):
        wid = lax.axis_index("s") * 2 + lax.axis_index("c")
        base = wid * ROWS_PER_SC
        iota = lax.iota(jnp.int32, 16)
        fzero16 = jnp.zeros((16,), jnp.float32)
        izero16 = jnp.zeros((16,), jnp.int32)
        BIG = jnp.int32(0x7FFFFFF)

        def _zero_row(i, _):
            Vt_v[pl.ds(i * 16, 16)] = fzero16
            Dt_v[pl.ds(i * 16, 16)] = izero16
            return 0
        lax.fori_loop(0, ROWS_PER_SC * CAP // 16, _zero_row, 0)

        def _zero_cnt(i, _):
            rcnt_v[pl.ds(i * 16, 16)] = izero16
            return 0
        lax.fori_loop(0, ROWS_PER_SC // 16, _zero_cnt, 0)

        h1 = pltpu.async_copy(bs_hbm.at[:, wid], eS_v, sem)
        h2 = pltpu.async_copy(bd_hbm.at[:, wid], eD_v, sem)
        h3 = pltpu.async_copy(bv_hbm.at[:, wid], eV_v, sem)
        h1.wait()
        h2.wait()
        h3.wait()

        def bucket_body(u, _):
            def vreg_body(g, _):
                s16 = eS_v[u, pl.ds(g * 16, 16)]
                valid = s16 < SENT
                key = jnp.where(valid, s16 - base, BIG)
                ks, perm, p, last = _vreg_group_slots(key, iota, tmp_v)
                vs = ks != BIG
                ksc = jnp.clip(ks, 0, ROWS_PER_SC - 1)
                cold = plsc.load_gather(rcnt_v, [ksc])
                slot = cold + p
                ok = vs & (slot < CAP)
                plsc.store_scatter(rcnt_v, [ksc], jnp.minimum(slot + 1, CAP),
                                   mask=last & vs)
                pabs = perm + g * 16
                dp = plsc.load_gather(eD_v, [jnp.full((16,), u, jnp.int32), pabs])
                vp = plsc.load_gather(eV_v, [jnp.full((16,), u, jnp.int32), pabs])
                flat = ksc * CAP + slot
                plsc.store_scatter(Dt_v, [flat], dp, mask=ok)
                plsc.store_scatter(Vt_v, [flat], vp, mask=ok)
                return 0
            lax.fori_loop(0, BCAP // 16, vreg_body, 0)
            return 0

        lax.fori_loop(0, NW, bucket_body, 0)
        pltpu.sync_copy(Vt_v, vt_hbm.at[pl.ds(base * CAP, ROWS_PER_SC * CAP)])
        pltpu.sync_copy(Dt_v, dt_hbm.at[pl.ds(base * CAP, ROWS_PER_SC * CAP)])

    return kern(bS, bD, bV)


def _rowproc_kernel(v_ref, d_ref, w3_ref, b3_ref, out_ref, c_ref, p_ref, r_ref):
    Vt = v_ref[...]
    Dt = d_ref[...]
    w3 = w3_ref[0, 0]
    b3 = b3_ref[0, 0]
    rowsum = jnp.sum(Vt, axis=0, keepdims=True)
    k = _leaky(rowsum * w3 + b3)
    aidx = lax.broadcasted_iota(jnp.int32, (CAP, 1), 0)

    for a in range(CAP):
        Da = Dt[a : a + 1, :]
        eq = Dt == Da
        c_ref[a : a + 1, :] = jnp.sum(jnp.where(eq, Vt, 0.0), axis=0, keepdims=True)
        seen = jnp.sum((eq & (aidx < a)).astype(jnp.int32), axis=0, keepdims=True)
        p_ref[a : a + 1, :] = (seen == 0).astype(jnp.int32)

    Ct = c_ref[...]
    Pt = p_ref[...] != 0
    for a in range(CAP):
        Da = Dt[a : a + 1, :]
        Ca = Ct[a : a + 1, :]
        beats = (Ct > Ca) | ((Ct == Ca) & (Dt < Da))
        r_ref[a : a + 1, :] = jnp.sum(
            jnp.where(beats & Pt, 1.0, 0.0), axis=0, keepdims=True)

    f = 1.5 - 0.5 * jnp.tanh(r_ref[...] - k)
    out_ref[...] = jnp.where(Pt, Ct * f, 0.0)


def _rowproc(VtabT, DtabT, w3, b3):
    n_pad = VtabT.shape[1]
    blk = 512
    return pl.pallas_call(
        _rowproc_kernel,
        grid=(n_pad // blk,),
        in_specs=[
            pl.BlockSpec((CAP, blk), lambda i: (0, i)),
            pl.BlockSpec((CAP, blk), lambda i: (0, i)),
            pl.BlockSpec((1, 1), lambda i: (0, 0)),
            pl.BlockSpec((1, 1), lambda i: (0, 0)),
        ],
        out_specs=pl.BlockSpec((CAP, blk), lambda i: (0, i)),
        out_shape=jax.ShapeDtypeStruct((CAP, n_pad), jnp.float32),
        scratch_shapes=[
            pltpu.VMEM((CAP, blk), jnp.float32),
            pltpu.VMEM((CAP, blk), jnp.int32),
            pltpu.VMEM((CAP, blk), jnp.float32),
        ],
    )(VtabT, DtabT, w3, b3.reshape(1, 1))


def _materialize(out_val, Dtab, N):
    G = CAP // 16

    mesh = plsc.VectorSubcoreMesh(core_axis_name="c", subcore_axis_name="s")

    @functools.partial(
        pl.kernel,
        out_type=jax.ShapeDtypeStruct((N, N), jnp.float32),
        mesh=mesh,
        compiler_params=_SC_PARAMS,
        scratch_types=[
            pltpu.VMEM((ROW_BLK, CAP), jnp.float32),
            pltpu.VMEM((ROW_BLK, CAP), jnp.int32),
            pltpu.VMEM((N,), jnp.float32),
            pltpu.VMEM((N,), jnp.float32),
            pltpu.VMEM((2, CAP), jnp.int32),
            pltpu.SemaphoreType.DMA,
            pltpu.SemaphoreType.DMA,
            pltpu.SemaphoreType.DMA,
        ],
    )
    def kern(val_hbm, col_hbm, out_hbm, val_v, col_v, buf0, buf1, pend, sem0, sem1, sem_in):
        wid = lax.axis_index("s") * 2 + lax.axis_index("c")
        base = wid * ROWS_PER_SC
        nrows = jnp.minimum(ROWS_PER_SC, N - base)
        nblk = nrows // ROW_BLK
        zeros16 = jnp.zeros((16,), jnp.float32)

        def _zero_init(i, _):
            buf0[pl.ds(i * 16, 16)] = zeros16
            buf1[pl.ds(i * 16, 16)] = zeros16
            return 0
        lax.fori_loop(0, N // 16, _zero_init, 0)

        def blk_body(b, first_flags):
            f0, f1 = first_flags
            pltpu.async_copy(
                val_hbm.at[pl.ds(base + b * ROW_BLK, ROW_BLK)], val_v, sem_in
            ).wait()
            pltpu.async_copy(
                col_hbm.at[pl.ds(base + b * ROW_BLK, ROW_BLK)], col_v, sem_in
            ).wait()

            def pair_body(p, flags):
                ff0, ff1 = flags

                def one(local_r, buf, sem, pslot, first):
                    @pl.when(jnp.logical_not(first))
                    def _():
                        pltpu.make_async_copy(buf, out_hbm.at[0], sem).wait()
                        for g in range(G):
                            idx = pend[pslot, pl.ds(g * 16, 16)]
                            plsc.store_scatter(buf, [idx], zeros16)
                    for g in range(G):
                        cols = col_v[local_r, pl.ds(g * 16, 16)]
                        vals = val_v[local_r, pl.ds(g * 16, 16)]
                        plsc.addupdate_scatter(buf, [cols], vals)
                        pend[pslot, pl.ds(g * 16, 16)] = cols
                    row = base + b * ROW_BLK + local_r
                    pltpu.async_copy(buf, out_hbm.at[row], sem)

                one(2 * p, buf0, sem0, 0, ff0)
                one(2 * p + 1, buf1, sem1, 1, ff1)
                return (jnp.bool_(False), jnp.bool_(False))

            return lax.fori_loop(0, ROW_BLK // 2, pair_body, (f0, f1))

        flags = lax.fori_loop(0, nblk, blk_body, (jnp.bool_(True), jnp.bool_(True)))
        @pl.when(jnp.logical_not(flags[0]))
        def _():
            pltpu.make_async_copy(buf0, out_hbm.at[0], sem0).wait()
        @pl.when(jnp.logical_not(flags[1]))
        def _():
            pltpu.make_async_copy(buf1, out_hbm.at[0], sem1).wait()

    return kern(out_val, Dtab)


def kernel(x, edge_index, W1, b1, W2, b2, w3, b3):
    src = edge_index[0].astype(jnp.int32)
    dst = edge_index[1].astype(jnp.int32)
    x_enc, y, yd = _encode(x, W1, b1, W2, b2)
    bS, bD, bV = _edge_bucket(y, yd, src, dst)
    n_pad = NW * ROWS_PER_SC
    vt_flat, dt_flat = _build_tables(bS, bD, bV, n_pad)
    Vtab = vt_flat.reshape(n_pad, CAP)
    Dtab = dt_flat.reshape(n_pad, CAP)
    out_valT = _rowproc(Vtab.T, Dtab.T, w3, b3)
    out = _materialize(out_valT.T, Dtab, x.shape[0])
    return out, x_enc
